# Initial kernel scaffold; baseline (speedup 1.0000x reference)
import jax
import jax.numpy as jnp
from jax.experimental import pallas as pl


def kernel(x, Wg, W1, b1, W2, b2):
    raise NotImplementedError("write your pallas kernel here")


# speedup vs baseline: 2.9496x; 2.9496x over previous
import functools

import jax
import jax.numpy as jnp
from jax.experimental import pallas as pl
from jax.experimental.pallas import tpu as pltpu

N = 2048
D = 768
H = 3072
E = 8
NEG_INF = -1e30


def _gate_kernel(xf_ref, wgt_ref, w_full_ref):
    logits = jax.lax.dot_general(
        xf_ref[...], wgt_ref[...], (((1,), (0,)), ((), ())),
        preferred_element_type=jnp.float32)
    col = jax.lax.broadcasted_iota(jnp.int32, (N, E), 1)
    m1 = jnp.max(logits, axis=1, keepdims=True)
    i1 = jnp.argmax(logits, axis=1).reshape(N, 1)
    masked = jnp.where(col == i1, NEG_INF, logits)
    m2 = jnp.max(masked, axis=1, keepdims=True)
    i2 = jnp.argmax(masked, axis=1).reshape(N, 1)
    e2 = jnp.exp(m2 - m1)
    denom = 1.0 + e2
    w1 = 1.0 / denom
    w2 = e2 / denom
    w_full_ref[...] = jnp.where(col == i1, w1, jnp.where(col == i2, w2, 0.0))


def _ffn_kernel(xt_ref, w1_ref, b1_ref, w2_ref, b2_ref, wt_ref, out_ref):
    e = pl.program_id(0)
    xt = xt_ref[...]
    acc = jnp.zeros((D, N), jnp.float32)
    for c in range(4):
        hc = H // 4
        h = jax.lax.dot_general(
            w1_ref[0, pl.ds(c * hc, hc), :], xt, (((1,), (0,)), ((), ())),
            preferred_element_type=jnp.float32)
        h = h + b1_ref[0, pl.ds(c * hc, hc), :]
        h = (0.5 * h * (1.0 + jax.lax.erf(h * 0.7071067811865476))
             ).astype(jnp.bfloat16)
        acc = acc + jax.lax.dot_general(
            w2_ref[0, :, pl.ds(c * hc, hc)], h, (((1,), (0,)), ((), ())),
            preferred_element_type=jnp.float32)
    acc = acc + b2_ref[0]
    contrib = acc * wt_ref[0]

    @pl.when(e == 0)
    def _():
        out_ref[...] = contrib

    @pl.when(e > 0)
    def _():
        out_ref[...] += contrib


def kernel(x, Wg, W1, b1, W2, b2):
    b, s, d = x.shape
    xf = x.reshape(N, D)

    w_full = pl.pallas_call(
        _gate_kernel,
        out_shape=jax.ShapeDtypeStruct((N, E), jnp.float32),
    )(xf, Wg.T)

    xt = xf.T.astype(jnp.bfloat16)
    w1b = W1.astype(jnp.bfloat16)
    w2b = W2.astype(jnp.bfloat16)
    b1r = b1.reshape(E, H, 1)
    b2r = b2.reshape(E, D, 1)
    wt = w_full.T.reshape(E, 1, N)

    out_t = pl.pallas_call(
        _ffn_kernel,
        grid=(E,),
        in_specs=[
            pl.BlockSpec((D, N), lambda e: (0, 0)),
            pl.BlockSpec((1, H, D), lambda e: (e, 0, 0)),
            pl.BlockSpec((1, H, 1), lambda e: (e, 0, 0)),
            pl.BlockSpec((1, D, H), lambda e: (e, 0, 0)),
            pl.BlockSpec((1, D, 1), lambda e: (e, 0, 0)),
            pl.BlockSpec((1, 1, N), lambda e: (e, 0, 0)),
        ],
        out_specs=pl.BlockSpec((D, N), lambda e: (0, 0)),
        out_shape=jax.ShapeDtypeStruct((D, N), jnp.float32),
    )(xt, w1b, b1r, w2b, b2r, wt)

    return out_t.T.reshape(b, s, d), w_full
